# baseline (speedup 1.0000x reference)
import functools

import jax
import jax.numpy as jnp
from jax import lax
from jax.experimental import pallas as pl
from jax.experimental.pallas import tpu as pltpu
from jax.experimental.pallas import tpu_sc as plsc

BATCH = 512
CW_DIM = 1024
Z_DIM = 512
H_DIM = 1024
C = 64
K = 512
D = 16

_BT = 64
_CG = 8

_SC_CORES = 1
_NW = 16 * _SC_CORES
_ROWS = BATCH * C
_BPW = _ROWS // _NW
_CHUNK = 128
_NCH = _BPW // _CHUNK


def _fused_body(x_ref, w1, b1, w2, b2, w3, b3, w4, b4, wg_ref, mask_ref,
                iota_ref,
                mu_ref, lv_ref, dist_ref, idx_ref, cw_s):
    ci = pl.program_id(1)
    f32 = jnp.float32

    @pl.when(ci == 0)
    def _mlp():
        h = jnp.maximum(jnp.dot(x_ref[...], w1[...], preferred_element_type=f32) + b1[...], 0.0)
        enc = jnp.dot(h, w2[...], preferred_element_type=f32) + b2[...]
        mu = enc[:, :Z_DIM]
        hd = jnp.maximum(jnp.dot(mu, w3[...], preferred_element_type=f32) + b3[...], 0.0)
        cw = jnp.dot(hd, w4[...], preferred_element_type=f32) + b4[...]
        mu_ref[...] = mu
        lv_ref[...] = enc[:, Z_DIM:]
        for g in range(C // _CG):
            cw_s[g] = cw[:, (g * _CG * D):((g + 1) * _CG * D)]

    cw_g = cw_s[ci]
    wg = wg_ref[0]
    a = (jnp.broadcast_to(cw_g[:, None, :], (_BT, _CG, _CG * D))
         .reshape(_BT * _CG, _CG * D) * mask_ref[...])
    cross = jnp.dot(a, wg, preferred_element_type=f32)
    x_sq = jnp.dot(a * a, jnp.ones((_CG * D, 1), f32),
                   preferred_element_type=f32)
    w2g = wg * wg
    b_sq = jnp.sum(w2g.reshape(_CG, D, K), axis=1)
    dist = (x_sq.reshape(_BT, _CG, 1) + b_sq[None, :, :]
            - 2.0 * cross.reshape(_BT, _CG, K))
    dist_ref[...] = dist
    m = jnp.min(dist, axis=2, keepdims=True)
    iota = iota_ref[...]
    first_min = jnp.min(jnp.where(dist == m, iota, float(K)), axis=2)
    crow = lax.broadcasted_iota(jnp.int32, (_BT, _CG), 1) + ci * _CG
    idx_val = first_min.astype(jnp.int32) + crow * K
    for k in range(C // _CG):
        @pl.when(ci == k)
        def _store_idx(k=k, idx_val=idx_val):
            idx_ref[:, (k * _CG):((k + 1) * _CG)] = idx_val


def _fused(x, W1, b1r, W2, b2r, W3, b3r, W4, b4r, wstack, mask):
    grid = (BATCH // _BT, C // _CG)
    const2 = lambda shape: pl.BlockSpec(shape, lambda bi, ci: (0, 0))
    call = pl.pallas_call(
        _fused_body,
        grid=grid,
        in_specs=[
            pl.BlockSpec((_BT, CW_DIM), lambda bi, ci: (bi, 0)),
            const2((CW_DIM, H_DIM)), const2((1, H_DIM)),
            const2((H_DIM, 2 * Z_DIM)), const2((1, 2 * Z_DIM)),
            const2((Z_DIM, H_DIM)), const2((1, H_DIM)),
            const2((H_DIM, CW_DIM)), const2((1, CW_DIM)),
            pl.BlockSpec((1, _CG * D, K), lambda bi, ci: (ci, 0, 0)),
            const2((_BT * _CG, _CG * D)),
            pl.BlockSpec((1, 1, K), lambda bi, ci: (0, 0, 0)),
        ],
        out_specs=[
            pl.BlockSpec((_BT, Z_DIM), lambda bi, ci: (bi, 0)),
            pl.BlockSpec((_BT, Z_DIM), lambda bi, ci: (bi, 0)),
            pl.BlockSpec((_BT, _CG, K), lambda bi, ci: (bi, ci, 0)),
            pl.BlockSpec((_BT, C), lambda bi, ci: (bi, 0)),
        ],
        out_shape=[
            jax.ShapeDtypeStruct((BATCH, Z_DIM), jnp.float32),
            jax.ShapeDtypeStruct((BATCH, Z_DIM), jnp.float32),
            jax.ShapeDtypeStruct((BATCH, C, K), jnp.float32),
            jax.ShapeDtypeStruct((BATCH, C), jnp.int32),
        ],
        scratch_shapes=[pltpu.VMEM((C // _CG, _BT, _CG * D), jnp.float32)],
    )
    iota = lax.broadcasted_iota(jnp.float32, (1, 1, K), 2)
    return call(x, W1, b1r, W2, b2r, W3, b3r, W4, b4r, wstack, mask, iota)


def _sc_gather(table, idx2d):
    mesh = plsc.VectorSubcoreMesh(core_axis_name="c", subcore_axis_name="s",
                                  num_cores=_SC_CORES)

    @functools.partial(
        pl.kernel,
        mesh=mesh,
        compiler_params=pltpu.CompilerParams(use_tc_tiling_on_sc=False),
        out_type=jax.ShapeDtypeStruct((_ROWS, D), jnp.float32),
        scratch_types=[
            pltpu.VMEM((_NCH, _CHUNK), jnp.int32),
            pltpu.VMEM((_BPW, D), jnp.float32),
            pltpu.SemaphoreType.DMA,
        ],
    )
    def gather_kernel(table_hbm, idx_hbm, out_hbm, idx_v, rows_v, sem):
        wid = lax.axis_index("s") * _SC_CORES + lax.axis_index("c")
        pltpu.sync_copy(idx_hbm.at[pl.ds(wid * _NCH, _NCH)], idx_v)
        copies = []
        for j in range(_NCH):
            copies.append(pltpu.async_copy(
                table_hbm.at[idx_v.at[j]],
                rows_v.at[pl.ds(j * _CHUNK, _CHUNK)], sem))
        for cp in copies:
            cp.wait()
        pltpu.sync_copy(rows_v, out_hbm.at[pl.ds(wid * _BPW, _BPW)])

    return gather_kernel(table, idx2d)


def kernel(x, W1, b1, W2, b2, W3, b3, W4, b4, codebook):
    cbt = codebook.transpose(0, 2, 1)
    wstack = cbt.reshape(C // _CG, _CG * D, K)
    mask = (lax.broadcasted_iota(jnp.int32, (_BT * _CG, _CG * D), 1) // D
            == lax.broadcasted_iota(jnp.int32, (_BT * _CG, _CG * D), 0) % _CG
            ).astype(jnp.float32)
    mu, lv, cw_dist, idx_bc = _fused(
        x, W1, b1.reshape(1, -1), W2, b2.reshape(1, -1),
        W3, b3.reshape(1, -1), W4, b4.reshape(1, -1), wstack, mask)
    flat_idx = idx_bc.reshape(_ROWS // _CHUNK, _CHUNK)
    table = codebook.reshape(C * K, D)
    closest = _sc_gather(table, flat_idx).reshape(BATCH, C * D)
    return (mu, lv, mu, cw_dist, closest)


# speedup vs baseline: 1.0983x; 1.0983x over previous
import functools

import jax
import jax.numpy as jnp
from jax import lax
from jax.experimental import pallas as pl
from jax.experimental.pallas import tpu as pltpu
from jax.experimental.pallas import tpu_sc as plsc

BATCH = 512
CW_DIM = 1024
Z_DIM = 512
H_DIM = 1024
C = 64
K = 512
D = 16

_BT = 128
_CG = 8

_SC_CORES = 1
_NW = 16 * _SC_CORES
_ROWS = BATCH * C
_BPW = _ROWS // _NW
_CHUNK = 128
_NCH = _BPW // _CHUNK


def _fused_body(x_ref, w1, b1, w2, b2, w3, b3, w4, b4, wg_ref, mask_ref,
                iota_ref,
                mu_ref, lv_ref, dist_ref, idx_ref, cw_s):
    ci = pl.program_id(1)
    f32 = jnp.float32

    @pl.when(ci == 0)
    def _mlp():
        h = jnp.maximum(jnp.dot(x_ref[...], w1[...], preferred_element_type=f32) + b1[...], 0.0)
        enc = jnp.dot(h, w2[...], preferred_element_type=f32) + b2[...]
        mu = enc[:, :Z_DIM]
        hd = jnp.maximum(jnp.dot(mu, w3[...], preferred_element_type=f32) + b3[...], 0.0)
        cw = jnp.dot(hd, w4[...], preferred_element_type=f32) + b4[...]
        mu_ref[...] = mu
        lv_ref[...] = enc[:, Z_DIM:]
        for g in range(C // _CG):
            cw_s[g] = cw[:, (g * _CG * D):((g + 1) * _CG * D)]

    cw_g = cw_s[ci]
    wg = wg_ref[0]
    a = (jnp.broadcast_to(cw_g[:, None, :], (_BT, _CG, _CG * D))
         .reshape(_BT * _CG, _CG * D) * mask_ref[...])
    cross = jnp.dot(a, wg, preferred_element_type=f32)
    x_sq = jnp.dot(a * a, jnp.ones((_CG * D, 1), f32),
                   preferred_element_type=f32)
    w2g = wg * wg
    b_sq = jnp.sum(w2g.reshape(_CG, D, K), axis=1)
    dist = (x_sq.reshape(_BT, _CG, 1) + b_sq[None, :, :]
            - 2.0 * cross.reshape(_BT, _CG, K))
    dist_ref[...] = dist
    m = jnp.min(dist, axis=2, keepdims=True)
    iota = iota_ref[...]
    first_min = jnp.min(jnp.where(dist == m, iota, float(K)), axis=2)
    crow = lax.broadcasted_iota(jnp.int32, (_BT, _CG), 1) + ci * _CG
    idx_val = first_min.astype(jnp.int32) + crow * K
    for k in range(C // _CG):
        @pl.when(ci == k)
        def _store_idx(k=k, idx_val=idx_val):
            idx_ref[:, (k * _CG):((k + 1) * _CG)] = idx_val


def _fused(x, W1, b1r, W2, b2r, W3, b3r, W4, b4r, wstack, mask):
    grid = (BATCH // _BT, C // _CG)
    const2 = lambda shape: pl.BlockSpec(shape, lambda bi, ci: (0, 0))
    call = pl.pallas_call(
        _fused_body,
        grid=grid,
        in_specs=[
            pl.BlockSpec((_BT, CW_DIM), lambda bi, ci: (bi, 0)),
            const2((CW_DIM, H_DIM)), const2((1, H_DIM)),
            const2((H_DIM, 2 * Z_DIM)), const2((1, 2 * Z_DIM)),
            const2((Z_DIM, H_DIM)), const2((1, H_DIM)),
            const2((H_DIM, CW_DIM)), const2((1, CW_DIM)),
            pl.BlockSpec((1, _CG * D, K), lambda bi, ci: (ci, 0, 0)),
            const2((_BT * _CG, _CG * D)),
            pl.BlockSpec((1, 1, K), lambda bi, ci: (0, 0, 0)),
        ],
        out_specs=[
            pl.BlockSpec((_BT, Z_DIM), lambda bi, ci: (bi, 0)),
            pl.BlockSpec((_BT, Z_DIM), lambda bi, ci: (bi, 0)),
            pl.BlockSpec((_BT, _CG, K), lambda bi, ci: (bi, ci, 0)),
            pl.BlockSpec((_BT, C), lambda bi, ci: (bi, 0)),
        ],
        out_shape=[
            jax.ShapeDtypeStruct((BATCH, Z_DIM), jnp.float32),
            jax.ShapeDtypeStruct((BATCH, Z_DIM), jnp.float32),
            jax.ShapeDtypeStruct((BATCH, C, K), jnp.float32),
            jax.ShapeDtypeStruct((BATCH, C), jnp.int32),
        ],
        scratch_shapes=[pltpu.VMEM((C // _CG, _BT, _CG * D), jnp.float32)],
    )
    iota = lax.broadcasted_iota(jnp.float32, (1, 1, K), 2)
    return call(x, W1, b1r, W2, b2r, W3, b3r, W4, b4r, wstack, mask, iota)


def _sc_gather(table, idx2d):
    mesh = plsc.VectorSubcoreMesh(core_axis_name="c", subcore_axis_name="s",
                                  num_cores=_SC_CORES)

    @functools.partial(
        pl.kernel,
        mesh=mesh,
        compiler_params=pltpu.CompilerParams(use_tc_tiling_on_sc=False),
        out_type=jax.ShapeDtypeStruct((_ROWS, D), jnp.float32),
        scratch_types=[
            pltpu.VMEM((_NCH, _CHUNK), jnp.int32),
            pltpu.VMEM((_BPW, D), jnp.float32),
            pltpu.SemaphoreType.DMA,
        ],
    )
    def gather_kernel(table_hbm, idx_hbm, out_hbm, idx_v, rows_v, sem):
        wid = lax.axis_index("s") * _SC_CORES + lax.axis_index("c")
        pltpu.sync_copy(idx_hbm.at[pl.ds(wid * _NCH, _NCH)], idx_v)
        copies = []
        for j in range(_NCH):
            copies.append(pltpu.async_copy(
                table_hbm.at[idx_v.at[j]],
                rows_v.at[pl.ds(j * _CHUNK, _CHUNK)], sem))
        for cp in copies:
            cp.wait()
        pltpu.sync_copy(rows_v, out_hbm.at[pl.ds(wid * _BPW, _BPW)])

    return gather_kernel(table, idx2d)


def kernel(x, W1, b1, W2, b2, W3, b3, W4, b4, codebook):
    cbt = codebook.transpose(0, 2, 1)
    wstack = cbt.reshape(C // _CG, _CG * D, K)
    mask = (lax.broadcasted_iota(jnp.int32, (_BT * _CG, _CG * D), 1) // D
            == lax.broadcasted_iota(jnp.int32, (_BT * _CG, _CG * D), 0) % _CG
            ).astype(jnp.float32)
    mu, lv, cw_dist, idx_bc = _fused(
        x, W1, b1.reshape(1, -1), W2, b2.reshape(1, -1),
        W3, b3.reshape(1, -1), W4, b4.reshape(1, -1), wstack, mask)
    flat_idx = idx_bc.reshape(_ROWS // _CHUNK, _CHUNK)
    table = codebook.reshape(C * K, D)
    closest = _sc_gather(table, flat_idx).reshape(BATCH, C * D)
    return (mu, lv, mu, cw_dist, closest)
